# baseline (speedup 1.0000x reference)
import jax
import jax.numpy as jnp
from jax.experimental import pallas as pl
from jax.experimental.pallas import tpu as pltpu

_N_BINS = 15
_ROWS = 16384
_COLS = 1000
_CBLK = 200
_G = _COLS // _CBLK
_NEG = -3.0e38


def _ece_kernel(x_ref, lab_ref, bnd_ref, out_ref, m_ref, s_ref, i_ref):
    i = pl.program_id(0)

    @pl.when(i == 0)
    def _init():
        m_ref[...] = jnp.full((1, _ROWS), _NEG, jnp.float32)
        s_ref[...] = jnp.zeros((1, _ROWS), jnp.float32)
        i_ref[...] = jnp.zeros((1, _ROWS), jnp.int32)

    x = x_ref[...]
    bm = jnp.max(x, axis=0, keepdims=True)
    col = jax.lax.broadcasted_iota(jnp.int32, x.shape, 0) + _CBLK * i
    bidx = jnp.min(jnp.where(x == bm, col, _COLS), axis=0, keepdims=True)

    m_old = m_ref[...]
    m_new = jnp.maximum(m_old, bm)
    ex = jnp.exp(x - m_new)
    ones = jnp.full((1, _CBLK), 1.0, jnp.float32)
    bsum = jax.lax.dot_general(
        ones,
        ex,
        (((1,), (0,)), ((), ())),
        preferred_element_type=jnp.float32,
    )
    s_ref[...] = s_ref[...] * jnp.exp(m_old - m_new) + bsum
    i_ref[...] = jnp.where(bm > m_old, bidx, i_ref[...])
    m_ref[...] = m_new

    @pl.when(i == _G - 1)
    def _finish():
        conf = 1.0 / s_ref[...]
        acc = (i_ref[...] == lab_ref[...]).astype(jnp.float32)
        lo = bnd_ref[:, 0:1]
        hi = bnd_ref[:, 1:2]
        in_bin = ((conf > lo) & (conf <= hi)).astype(jnp.float32)

        def lane_sum(v):
            return jax.lax.dot_general(
                in_bin,
                v,
                (((1,), (1,)), ((), ())),
                preferred_element_type=jnp.float32,
            ).reshape(_N_BINS)

        cnt = lane_sum(jnp.full((1, _ROWS), 1.0, jnp.float32))
        cs = lane_sum(conf)
        as_ = lane_sum(acc)
        prop = cnt / float(_ROWS)
        denom = jnp.maximum(cnt, 1.0)
        gaps = jnp.where(
            cnt > 0.0, jnp.abs(cs / denom - as_ / denom) * prop, 0.0
        )
        out_ref[...] = jnp.sum(gaps).reshape(1, 1)


@jax.jit
def _ece(logits, labels):
    xt = logits.T
    labels2 = labels.astype(jnp.int32).reshape(1, _ROWS)
    bb = jnp.linspace(0.0, 1.0, _N_BINS + 1)
    bounds = jnp.stack([bb[:-1], bb[1:]], axis=1)
    out = pl.pallas_call(
        _ece_kernel,
        grid=(_G,),
        in_specs=[
            pl.BlockSpec((_CBLK, _ROWS), lambda i: (i, 0)),
            pl.BlockSpec((1, _ROWS), lambda i: (0, 0)),
            pl.BlockSpec((_N_BINS, 2), lambda i: (0, 0)),
        ],
        out_specs=pl.BlockSpec((1, 1), lambda i: (0, 0)),
        out_shape=jax.ShapeDtypeStruct((1, 1), jnp.float32),
        scratch_shapes=[
            pltpu.VMEM((1, _ROWS), jnp.float32),
            pltpu.VMEM((1, _ROWS), jnp.float32),
            pltpu.VMEM((1, _ROWS), jnp.int32),
        ],
    )(xt, labels2, bounds)
    return out.reshape(1)


def kernel(logits, labels):
    return _ece(logits, labels)
